# baseline (speedup 1.0000x reference)
import jax
import jax.numpy as jnp
from jax.experimental import pallas as pl
from jax.experimental.pallas import tpu as pltpu

_N = 50000
_F = 256
_K = 2048
_ROWS = 32
_PAD = _ROWS * _K
_BN = 512


def _score_kernel(e_ref, m_ref, s_ref, o_ref):
    pid = pl.program_id(0)
    w = s_ref[...]
    inv = jax.lax.rsqrt(jnp.sum(w * w))
    s = jnp.dot(e_ref[...], w, preferred_element_type=jnp.float32) * inv
    s = s + m_ref[...]
    row = pid * _K + jax.lax.broadcasted_iota(jnp.int32, (_K, 1), 0)
    o_ref[...] = jnp.where(row < _N, s, -jnp.inf)


def _stage(v, ix, d, want_desc):
    c = jax.lax.broadcasted_iota(jnp.int32, v.shape, 1)
    low = (c & d) == 0
    pv = jnp.where(low, jnp.roll(v, -d, axis=1), jnp.roll(v, d, axis=1))
    pi = jnp.where(low, jnp.roll(ix, -d, axis=1), jnp.roll(ix, d, axis=1))
    gt = (v > pv) | ((v == pv) & (ix < pi))
    keep = gt == (want_desc == low)
    return jnp.where(keep, v, pv), jnp.where(keep, ix, pi)


def _topk_kernel(s_ref, v_ref, i_ref, g_ref):
    v = s_ref[...]
    r_io = jax.lax.broadcasted_iota(jnp.int32, (_ROWS, _K), 0)
    c_io = jax.lax.broadcasted_iota(jnp.int32, (_ROWS, _K), 1)
    ix = r_io * _K + c_io

    row_desc = r_io < (_ROWS // 2)
    k = 2
    while k <= _K:
        d = k // 2
        while d >= 1:
            blk = (c_io & k) == 0
            v, ix = _stage(v, ix, d, blk == row_desc)
            d //= 2
        k *= 2

    rows = _ROWS
    while rows > 1:
        h = rows // 2
        a_v, b_v = v[:h], v[h:]
        a_i, b_i = ix[:h], ix[h:]
        gt = (a_v > b_v) | ((a_v == b_v) & (a_i < b_i))
        v = jnp.where(gt, a_v, b_v)
        ix = jnp.where(gt, a_i, b_i)
        if h == 1:
            rd = jnp.full((1, _K), True)
        else:
            rd = jax.lax.broadcasted_iota(jnp.int32, (h, _K), 0) < (h // 2)
        d = _K // 2
        while d >= 1:
            v, ix = _stage(v, ix, d, rd)
            d //= 2
        rows = h

    v_ref[...] = v
    i_ref[...] = ix
    g_ref[...] = jnp.tanh(v)


def _gather_kernel(i_ref, g_ref, e_ref, o_ref):
    pid = pl.program_id(0)

    @pl.when(pid == 0)
    def _init():
        o_ref[...] = jnp.zeros_like(o_ref)

    idx = i_ref[...]
    gate = g_ref[...]
    rows = pid * _BN + jax.lax.broadcasted_iota(jnp.int32, (_BN, _K), 0)
    oh = jnp.where(rows == idx, gate, 0.0)
    acc = jax.lax.dot_general(
        e_ref[...], oh, (((0,), (0,)), ((), ())),
        preferred_element_type=jnp.float32)
    o_ref[...] += acc


def kernel(embeddings, mask, scorer):
    e_pad = jnp.pad(embeddings, ((0, _PAD - _N), (0, 0)))
    m_pad = jnp.pad(mask, ((0, _PAD - _N), (0, 0)))

    scores = pl.pallas_call(
        _score_kernel,
        grid=(_ROWS,),
        in_specs=[
            pl.BlockSpec((_K, _F), lambda i: (i, 0)),
            pl.BlockSpec((_K, 1), lambda i: (i, 0)),
            pl.BlockSpec((_F, 1), lambda i: (0, 0)),
        ],
        out_specs=pl.BlockSpec((_K, 1), lambda i: (i, 0)),
        out_shape=jax.ShapeDtypeStruct((_PAD, 1), jnp.float32),
    )(e_pad, m_pad, scorer)

    vals, idx, gate = pl.pallas_call(
        _topk_kernel,
        out_shape=(
            jax.ShapeDtypeStruct((1, _K), jnp.float32),
            jax.ShapeDtypeStruct((1, _K), jnp.int32),
            jax.ShapeDtypeStruct((1, _K), jnp.float32),
        ),
    )(scores.reshape(_ROWS, _K))

    out = pl.pallas_call(
        _gather_kernel,
        grid=(_PAD // _BN,),
        in_specs=[
            pl.BlockSpec((1, _K), lambda i: (0, 0)),
            pl.BlockSpec((1, _K), lambda i: (0, 0)),
            pl.BlockSpec((_BN, _F), lambda i: (i, 0)),
        ],
        out_specs=pl.BlockSpec((_F, _K), lambda i: (0, 0)),
        out_shape=jax.ShapeDtypeStruct((_F, _K), jnp.float32),
    )(idx, gate, e_pad)
    return out


# speedup vs baseline: 1.1277x; 1.1277x over previous
import jax
import jax.numpy as jnp
from jax.experimental import pallas as pl
from jax.experimental.pallas import tpu as pltpu

_N = 50000
_F = 256
_K = 2048
_ROWS = 32
_PAD = _ROWS * _K
_BN = 1024


def _score_kernel(e_ref, m_ref, s_ref, o_ref):
    pid = pl.program_id(0)
    w = s_ref[...]
    inv = jax.lax.rsqrt(jnp.sum(w * w))
    s = jnp.dot(e_ref[...], w, preferred_element_type=jnp.float32) * inv
    s = s + m_ref[...]
    row = pid * _K + jax.lax.broadcasted_iota(jnp.int32, (_K, 1), 0)
    o_ref[...] = jnp.where(row < _N, s, -jnp.inf)


def _stage(v, ix, d, want_desc):
    c = jax.lax.broadcasted_iota(jnp.int32, v.shape, 1)
    low = (c & d) == 0
    pv = jnp.where(low, jnp.roll(v, -d, axis=1), jnp.roll(v, d, axis=1))
    pi = jnp.where(low, jnp.roll(ix, -d, axis=1), jnp.roll(ix, d, axis=1))
    gt = (v > pv) | ((v == pv) & (ix < pi))
    keep = gt == (want_desc == low)
    return jnp.where(keep, v, pv), jnp.where(keep, ix, pi)


def _topk_kernel(s_ref, v_ref, i_ref, g_ref):
    v = s_ref[...]
    r_io = jax.lax.broadcasted_iota(jnp.int32, (_ROWS, _K), 0)
    c_io = jax.lax.broadcasted_iota(jnp.int32, (_ROWS, _K), 1)
    ix = r_io * _K + c_io

    row_desc = r_io < (_ROWS // 2)
    k = 2
    while k <= _K:
        d = k // 2
        while d >= 1:
            blk = (c_io & k) == 0
            v, ix = _stage(v, ix, d, blk == row_desc)
            d //= 2
        k *= 2

    rows = _ROWS
    while rows > 1:
        h = rows // 2
        a_v, b_v = v[:h], v[h:]
        a_i, b_i = ix[:h], ix[h:]
        gt = (a_v > b_v) | ((a_v == b_v) & (a_i < b_i))
        v = jnp.where(gt, a_v, b_v)
        ix = jnp.where(gt, a_i, b_i)
        if h == 1:
            rd = jnp.full((1, _K), True)
        else:
            rd = jax.lax.broadcasted_iota(jnp.int32, (h, _K), 0) < (h // 2)
        d = _K // 2
        while d >= 1:
            v, ix = _stage(v, ix, d, rd)
            d //= 2
        rows = h

    v_ref[...] = v
    i_ref[...] = ix
    g_ref[...] = jnp.tanh(v)


def _gather_kernel(i_ref, g_ref, e_ref, o_ref):
    pid = pl.program_id(0)

    @pl.when(pid == 0)
    def _init():
        o_ref[...] = jnp.zeros_like(o_ref)

    idx = i_ref[...]
    gate = g_ref[...]
    rows = pid * _BN + jax.lax.broadcasted_iota(jnp.int32, (_BN, _K), 0)
    oh = jnp.where(rows == idx, gate, 0.0).astype(jnp.bfloat16)
    acc = jax.lax.dot_general(
        e_ref[...].astype(jnp.bfloat16), oh, (((0,), (0,)), ((), ())),
        preferred_element_type=jnp.float32)
    o_ref[...] += acc


def kernel(embeddings, mask, scorer):
    e_pad = jnp.pad(embeddings, ((0, _PAD - _N), (0, 0)))
    m_pad = jnp.pad(mask, ((0, _PAD - _N), (0, 0)))

    scores = pl.pallas_call(
        _score_kernel,
        grid=(_ROWS,),
        in_specs=[
            pl.BlockSpec((_K, _F), lambda i: (i, 0)),
            pl.BlockSpec((_K, 1), lambda i: (i, 0)),
            pl.BlockSpec((_F, 1), lambda i: (0, 0)),
        ],
        out_specs=pl.BlockSpec((_K, 1), lambda i: (i, 0)),
        out_shape=jax.ShapeDtypeStruct((_PAD, 1), jnp.float32),
    )(e_pad, m_pad, scorer)

    vals, idx, gate = pl.pallas_call(
        _topk_kernel,
        out_shape=(
            jax.ShapeDtypeStruct((1, _K), jnp.float32),
            jax.ShapeDtypeStruct((1, _K), jnp.int32),
            jax.ShapeDtypeStruct((1, _K), jnp.float32),
        ),
    )(scores.reshape(_ROWS, _K))

    out = pl.pallas_call(
        _gather_kernel,
        grid=(_PAD // _BN,),
        in_specs=[
            pl.BlockSpec((1, _K), lambda i: (0, 0)),
            pl.BlockSpec((1, _K), lambda i: (0, 0)),
            pl.BlockSpec((_BN, _F), lambda i: (i, 0)),
        ],
        out_specs=pl.BlockSpec((_F, _K), lambda i: (0, 0)),
        out_shape=jax.ShapeDtypeStruct((_F, _K), jnp.float32),
    )(idx, gate, e_pad)
    return out
